# Initial kernel scaffold; baseline (speedup 1.0000x reference)
import jax
import jax.numpy as jnp
from jax.experimental import pallas as pl


def kernel(action_indices, table):
    raise NotImplementedError("write your pallas kernel here")


# speedup vs baseline: 6.2391x; 6.2391x over previous
import functools

import jax
import jax.numpy as jnp
from jax import lax
from jax.experimental import pallas as pl
from jax.experimental.pallas import tpu as pltpu
from jax.experimental.pallas import tpu_sc as plsc

D = 256
NC = 2
NS = 16
NW = NC * NS
CH = 128
NBUF = 2


def _build(B):
    bpw = B // NW
    nchunk = bpw // CH
    mesh = plsc.VectorSubcoreMesh(core_axis_name="c", subcore_axis_name="s")

    @functools.partial(
        pl.kernel,
        mesh=mesh,
        out_type=jax.ShapeDtypeStruct((B, D), jnp.float32),
        scratch_types=[
            pltpu.VMEM((nchunk, CH), jnp.int32),
            pltpu.VMEM((NBUF, CH, D), jnp.float32),
        ] + [pltpu.SemaphoreType.DMA] * (2 * NBUF),
    )
    def emb_kernel(idx_hbm, table_hbm, out_hbm, idx_v, rows_v, *sems):
        gsem = sems[:NBUF]
        osem = sems[NBUF:]
        wid = lax.axis_index("s") * NC + lax.axis_index("c")
        base = wid * bpw

        pltpu.sync_copy(idx_hbm.at[wid], idx_v)

        def gather_start(g, b):
            pltpu.async_copy(table_hbm.at[idx_v.at[g]], rows_v.at[b], gsem[b])

        def gather_wait(g, b):
            pltpu.make_async_copy(
                table_hbm.at[idx_v.at[g]], rows_v.at[b], gsem[b]).wait()

        def out_start(g, b):
            pltpu.async_copy(
                rows_v.at[b], out_hbm.at[pl.ds(base + g * CH, CH)], osem[b])

        def out_wait(g, b):
            pltpu.make_async_copy(
                rows_v.at[b], out_hbm.at[pl.ds(base + g * CH, CH)],
                osem[b]).wait()

        for b in range(NBUF):
            gather_start(b, b)

        def outer(i, carry):
            g0 = i * NBUF
            for b in range(NBUF):
                g = g0 + b
                gather_wait(g, b)
                out_start(g, b)
                out_wait(g, b)
                gather_start(g + NBUF, b)
            return carry

        lax.fori_loop(0, nchunk // NBUF - 1, outer, 0)

        for b in range(NBUF):
            g = nchunk - NBUF + b
            gather_wait(g, b)
            out_start(g, b)
            out_wait(g, b)

    return emb_kernel


@jax.jit
def kernel(action_indices, table):
    batch, seq = action_indices.shape
    b_total = batch * seq
    idx = action_indices.astype(jnp.int32).reshape(
        NW, (b_total // NW) // CH, CH)
    out = _build(b_total)(idx, table)
    return out.reshape(batch, seq, D)
